# baseline (speedup 1.0000x reference)
import functools

import jax
import jax.numpy as jnp
from jax import lax
from jax.experimental import pallas as pl
from jax.experimental.pallas import tpu as pltpu
from jax.experimental.pallas import tpu_sc as plsc

E = 8
D = 1024
N = 32768
BLK = 2048

_NC = 2
_NS = 16
_NW = _NC * _NS
_TOK_PER_W = N // _NW
_L = 16


def _affine_body(y_ref, s_ref, b_ref, x_ref, out_ref):
    y = y_ref[0, 0, :]
    classes = lax.broadcasted_iota(jnp.int32, (1, E), 1)
    onehot = (y[:, None] == classes).astype(jnp.float32)
    es = jnp.exp(s_ref[...])
    es_tok = jnp.dot(onehot, es, preferred_element_type=jnp.float32)
    b_tok = jnp.dot(onehot, b_ref[...], preferred_element_type=jnp.float32)
    out_ref[...] = x_ref[...] * es_tok + b_tok


def _reg_gather(v, idx):
    dn = lax.GatherDimensionNumbers(
        offset_dims=(), collapsed_slice_dims=(0,), start_index_map=(0,)
    )
    return lax.gather(
        v,
        idx[:, None],
        dn,
        slice_sizes=(1,),
        mode=lax.GatherScatterMode.PROMISE_IN_BOUNDS,
    )


def _logdet_sc(s_hbm, y_hbm, ld_hbm, s_v, y_v, ld_v, cls_v):
    wid = lax.axis_index("s") * _NC + lax.axis_index("c")
    base = wid * _TOK_PER_W
    pltpu.sync_copy(s_hbm, s_v)
    pltpu.sync_copy(y_hbm.at[pl.ds(base, _TOK_PER_W)], y_v)

    lane = lax.iota(jnp.int32, _L)
    cls = jnp.zeros((_L,), jnp.float32)
    for e in range(E):
        acc = jnp.zeros((_L,), jnp.float32)
        for j in range(D // _L):
            acc = acc + s_v[e, pl.ds(j * _L, _L)]
        for sh in (1, 2, 4, 8):
            perm = jnp.bitwise_xor(lane, sh)
            acc = acc + _reg_gather(acc, perm)
        cls = jnp.where(lane == e, acc, cls)
    cls_v[...] = cls

    def body(g, _):
        yv = y_v[pl.ds(g * _L, _L)]
        ld_v[pl.ds(g * _L, _L)] = plsc.load_gather(cls_v, [yv])
        return ()

    lax.fori_loop(0, _TOK_PER_W // _L, body, (), unroll=8)
    pltpu.sync_copy(ld_v, ld_hbm.at[pl.ds(base, _TOK_PER_W)])


@functools.partial(
    pl.kernel,
    out_type=jax.ShapeDtypeStruct((N,), jnp.float32),
    mesh=plsc.VectorSubcoreMesh(core_axis_name="c", subcore_axis_name="s"),
    scratch_types=[
        pltpu.VMEM((E, D), jnp.float32),
        pltpu.VMEM((_TOK_PER_W,), jnp.int32),
        pltpu.VMEM((_TOK_PER_W,), jnp.float32),
        pltpu.VMEM((_L,), jnp.float32),
    ],
    compiler_params=pltpu.CompilerParams(needs_layout_passes=False),
)
def _logdet_kernel(s_hbm, y_hbm, ld_hbm, s_v, y_v, ld_v, cls_v):
    _logdet_sc(s_hbm, y_hbm, ld_hbm, s_v, y_v, ld_v, cls_v)


@jax.jit
def kernel(x, y_idx, s, b):
    nblk = N // BLK
    y32 = y_idx.astype(jnp.int32)
    y3 = y32.reshape(nblk, 1, BLK)
    out = pl.pallas_call(
        _affine_body,
        grid=(nblk,),
        in_specs=[
            pl.BlockSpec((1, 1, BLK), lambda i: (i, 0, 0)),
            pl.BlockSpec((E, D), lambda i: (0, 0)),
            pl.BlockSpec((E, D), lambda i: (0, 0)),
            pl.BlockSpec((BLK, D), lambda i: (i, 0)),
        ],
        out_specs=pl.BlockSpec((BLK, D), lambda i: (i, 0)),
        out_shape=jax.ShapeDtypeStruct((N, D), jnp.float32),
        compiler_params=pltpu.CompilerParams(
            dimension_semantics=("arbitrary",),
        ),
    )(y3, s, b, x)
    ld = _logdet_kernel(s, y32)
    return out, ld


# speedup vs baseline: 1.0010x; 1.0010x over previous
import functools

import jax
import jax.numpy as jnp
from jax import lax
from jax.experimental import pallas as pl
from jax.experimental.pallas import tpu as pltpu
from jax.experimental.pallas import tpu_sc as plsc

E = 8
D = 1024
N = 32768
BLK = 2048

_NC = 2
_NS = 16
_NW = _NC * _NS
_TOK_PER_W = N // _NW
_L = 16


def _affine_body(y_ref, s_ref, b_ref, x_ref, out_ref):
    y = y_ref[0, 0, :]
    classes = lax.broadcasted_iota(jnp.int32, (1, E), 1)
    onehot = (y[:, None] == classes).astype(jnp.float32)
    es = jnp.exp(s_ref[...])
    es_tok = jnp.dot(onehot, es, preferred_element_type=jnp.float32)
    b_tok = jnp.dot(onehot, b_ref[...], preferred_element_type=jnp.float32)
    out_ref[...] = x_ref[...] * es_tok + b_tok


def _reg_gather(v, idx):
    dn = lax.GatherDimensionNumbers(
        offset_dims=(), collapsed_slice_dims=(0,), start_index_map=(0,)
    )
    return lax.gather(
        v,
        idx[:, None],
        dn,
        slice_sizes=(1,),
        mode=lax.GatherScatterMode.PROMISE_IN_BOUNDS,
    )


def _logdet_sc(s_hbm, y_hbm, ld_hbm, s_v, y_v, ld_v, cls_v):
    wid = lax.axis_index("s") * _NC + lax.axis_index("c")
    base = wid * _TOK_PER_W
    pltpu.sync_copy(s_hbm, s_v)
    pltpu.sync_copy(y_hbm.at[pl.ds(base, _TOK_PER_W)], y_v)

    lane = lax.iota(jnp.int32, _L)
    cls = jnp.zeros((_L,), jnp.float32)
    for e in range(E):
        acc = jnp.zeros((_L,), jnp.float32)
        for j in range(D // _L):
            acc = acc + s_v[e, pl.ds(j * _L, _L)]
        for sh in (1, 2, 4, 8):
            perm = jnp.bitwise_xor(lane, sh)
            acc = acc + _reg_gather(acc, perm)
        cls = jnp.where(lane == e, acc, cls)
    cls_v[...] = cls

    def body(g, _):
        yv = y_v[pl.ds(g * _L, _L)]
        ld_v[pl.ds(g * _L, _L)] = plsc.load_gather(cls_v, [yv])
        return ()

    lax.fori_loop(0, _TOK_PER_W // _L, body, (), unroll=8)
    pltpu.sync_copy(ld_v, ld_hbm.at[pl.ds(base, _TOK_PER_W)])


@functools.partial(
    pl.kernel,
    out_type=jax.ShapeDtypeStruct((N,), jnp.float32),
    mesh=plsc.VectorSubcoreMesh(core_axis_name="c", subcore_axis_name="s"),
    scratch_types=[
        pltpu.VMEM((E, D), jnp.float32),
        pltpu.VMEM((_TOK_PER_W,), jnp.int32),
        pltpu.VMEM((_TOK_PER_W,), jnp.float32),
        pltpu.VMEM((_L,), jnp.float32),
    ],
    compiler_params=pltpu.CompilerParams(
        needs_layout_passes=False, skip_device_barrier=True
    ),
)
def _logdet_kernel(s_hbm, y_hbm, ld_hbm, s_v, y_v, ld_v, cls_v):
    _logdet_sc(s_hbm, y_hbm, ld_hbm, s_v, y_v, ld_v, cls_v)


@jax.jit
def kernel(x, y_idx, s, b):
    nblk = N // BLK
    y32 = y_idx.astype(jnp.int32)
    y3 = y32.reshape(nblk, 1, BLK)
    out = pl.pallas_call(
        _affine_body,
        grid=(nblk,),
        in_specs=[
            pl.BlockSpec((1, 1, BLK), lambda i: (i, 0, 0)),
            pl.BlockSpec((E, D), lambda i: (0, 0)),
            pl.BlockSpec((E, D), lambda i: (0, 0)),
            pl.BlockSpec((BLK, D), lambda i: (i, 0)),
        ],
        out_specs=pl.BlockSpec((BLK, D), lambda i: (i, 0)),
        out_shape=jax.ShapeDtypeStruct((N, D), jnp.float32),
        compiler_params=pltpu.CompilerParams(
            dimension_semantics=("arbitrary",),
        ),
    )(y3, s, b, x)
    ld = _logdet_kernel(s, y32)
    return out, ld


# speedup vs baseline: 1.0256x; 1.0246x over previous
import functools

import jax
import jax.numpy as jnp
from jax import lax
from jax.experimental import pallas as pl
from jax.experimental.pallas import tpu as pltpu
from jax.experimental.pallas import tpu_sc as plsc

E = 8
D = 1024
N = 32768
BLK = 2048

_NC = 2
_NS = 16
_NW = _NC * _NS
_TOK_PER_W = N // _NW
_L = 16


def _affine_body(y_ref, s_ref, b_ref, x_ref, out_ref):
    y = y_ref[0, 0, :]
    classes = lax.broadcasted_iota(jnp.int32, (1, E), 1)
    onehot = (y[:, None] == classes).astype(jnp.float32)
    es = jnp.exp(s_ref[...])
    es_tok = jnp.dot(onehot, es, preferred_element_type=jnp.float32)
    b_tok = jnp.dot(onehot, b_ref[...], preferred_element_type=jnp.float32)
    out_ref[...] = x_ref[...] * es_tok + b_tok


def _reg_gather(v, idx):
    dn = lax.GatherDimensionNumbers(
        offset_dims=(), collapsed_slice_dims=(0,), start_index_map=(0,)
    )
    return lax.gather(
        v,
        idx[:, None],
        dn,
        slice_sizes=(1,),
        mode=lax.GatherScatterMode.PROMISE_IN_BOUNDS,
    )


def _logdet_sc(s_hbm, y_hbm, ld_hbm, s_v, y_v, ld_v, cls_v):
    wid = lax.axis_index("s") * _NC + lax.axis_index("c")
    base = wid * _TOK_PER_W
    pltpu.sync_copy(s_hbm, s_v)
    pltpu.sync_copy(y_hbm.at[pl.ds(base, _TOK_PER_W)], y_v)

    lane = lax.iota(jnp.int32, _L)
    cls = jnp.zeros((_L,), jnp.float32)
    for e in range(E):
        def rsum(j, acc):
            return acc + s_v[e, pl.ds(j * _L, _L)]

        acc = lax.fori_loop(0, D // _L, rsum, jnp.zeros((_L,), jnp.float32))
        for sh in (1, 2, 4, 8):
            perm = jnp.bitwise_xor(lane, sh)
            acc = acc + _reg_gather(acc, perm)
        cls = jnp.where(lane == e, acc, cls)
    cls_v[...] = cls

    def body(g, _):
        yv = y_v[pl.ds(g * _L, _L)]
        ld_v[pl.ds(g * _L, _L)] = plsc.load_gather(cls_v, [yv])
        return ()

    lax.fori_loop(0, _TOK_PER_W // _L, body, ())
    pltpu.sync_copy(ld_v, ld_hbm.at[pl.ds(base, _TOK_PER_W)])


@functools.partial(
    pl.kernel,
    out_type=jax.ShapeDtypeStruct((N,), jnp.float32),
    mesh=plsc.VectorSubcoreMesh(core_axis_name="c", subcore_axis_name="s"),
    scratch_types=[
        pltpu.VMEM((E, D), jnp.float32),
        pltpu.VMEM((_TOK_PER_W,), jnp.int32),
        pltpu.VMEM((_TOK_PER_W,), jnp.float32),
        pltpu.VMEM((_L,), jnp.float32),
    ],
    compiler_params=pltpu.CompilerParams(
        needs_layout_passes=False, skip_device_barrier=True
    ),
)
def _logdet_kernel(s_hbm, y_hbm, ld_hbm, s_v, y_v, ld_v, cls_v):
    _logdet_sc(s_hbm, y_hbm, ld_hbm, s_v, y_v, ld_v, cls_v)


@jax.jit
def kernel(x, y_idx, s, b):
    nblk = N // BLK
    y32 = y_idx.astype(jnp.int32)
    y3 = y32.reshape(nblk, 1, BLK)
    out = pl.pallas_call(
        _affine_body,
        grid=(nblk,),
        in_specs=[
            pl.BlockSpec((1, 1, BLK), lambda i: (i, 0, 0)),
            pl.BlockSpec((E, D), lambda i: (0, 0)),
            pl.BlockSpec((E, D), lambda i: (0, 0)),
            pl.BlockSpec((BLK, D), lambda i: (i, 0)),
        ],
        out_specs=pl.BlockSpec((BLK, D), lambda i: (i, 0)),
        out_shape=jax.ShapeDtypeStruct((N, D), jnp.float32),
        compiler_params=pltpu.CompilerParams(
            dimension_semantics=("arbitrary",),
            vmem_limit_bytes=110 * 1024 * 1024,
        ),
    )(y3, s, b, x)
    ld = _logdet_kernel(s, y32)
    return out, ld


# speedup vs baseline: 1.0312x; 1.0055x over previous
import functools

import jax
import jax.numpy as jnp
from jax import lax
from jax.experimental import pallas as pl
from jax.experimental.pallas import tpu as pltpu
from jax.experimental.pallas import tpu_sc as plsc

E = 8
D = 1024
N = 32768
BLK = 2048

_NC = 2
_NS = 16
_NW = _NC * _NS
_TOK_PER_W = N // _NW
_L = 16


def _affine_body(y_ref, s_ref, b_ref, x_ref, out_ref):
    y = y_ref[0, 0, :]
    classes = lax.broadcasted_iota(jnp.int32, (1, E), 1)
    onehot = (y[:, None] == classes).astype(jnp.float32)
    es = jnp.exp(s_ref[...])
    es_tok = jnp.dot(onehot, es, preferred_element_type=jnp.float32)
    b_tok = jnp.dot(onehot, b_ref[...], preferred_element_type=jnp.float32)
    out_ref[...] = x_ref[...] * es_tok + b_tok


def _reg_gather(v, idx):
    dn = lax.GatherDimensionNumbers(
        offset_dims=(), collapsed_slice_dims=(0,), start_index_map=(0,)
    )
    return lax.gather(
        v,
        idx[:, None],
        dn,
        slice_sizes=(1,),
        mode=lax.GatherScatterMode.PROMISE_IN_BOUNDS,
    )


def _logdet_sc(s_hbm, y_hbm, ld_hbm, s_v, y_v, ld_v, cls_v):
    wid = lax.axis_index("s") * _NC + lax.axis_index("c")
    base = wid * _TOK_PER_W
    pltpu.sync_copy(s_hbm, s_v)
    pltpu.sync_copy(y_hbm.at[pl.ds(base, _TOK_PER_W)], y_v)

    lane = lax.iota(jnp.int32, _L)

    def per_class(e, cls):
        def rsum(j, acc):
            return acc + s_v[e, pl.ds(j * _L, _L)]

        acc = lax.fori_loop(0, D // _L, rsum, jnp.zeros((_L,), jnp.float32))
        for sh in (1, 2, 4, 8):
            perm = jnp.bitwise_xor(lane, sh)
            acc = acc + _reg_gather(acc, perm)
        return jnp.where(lane == e, acc, cls)

    cls_v[...] = lax.fori_loop(0, E, per_class, jnp.zeros((_L,), jnp.float32))

    def body(g, _):
        yv = y_v[pl.ds(g * _L, _L)]
        ld_v[pl.ds(g * _L, _L)] = plsc.load_gather(cls_v, [yv])
        return ()

    lax.fori_loop(0, _TOK_PER_W // _L, body, ())
    pltpu.sync_copy(ld_v, ld_hbm.at[pl.ds(base, _TOK_PER_W)])


@functools.partial(
    pl.kernel,
    out_type=jax.ShapeDtypeStruct((N,), jnp.float32),
    mesh=plsc.VectorSubcoreMesh(core_axis_name="c", subcore_axis_name="s"),
    scratch_types=[
        pltpu.VMEM((E, D), jnp.float32),
        pltpu.VMEM((_TOK_PER_W,), jnp.int32),
        pltpu.VMEM((_TOK_PER_W,), jnp.float32),
        pltpu.VMEM((_L,), jnp.float32),
    ],
    compiler_params=pltpu.CompilerParams(
        needs_layout_passes=False, skip_device_barrier=True
    ),
)
def _logdet_kernel(s_hbm, y_hbm, ld_hbm, s_v, y_v, ld_v, cls_v):
    _logdet_sc(s_hbm, y_hbm, ld_hbm, s_v, y_v, ld_v, cls_v)


@jax.jit
def kernel(x, y_idx, s, b):
    nblk = N // BLK
    y32 = y_idx.astype(jnp.int32)
    y3 = y32.reshape(nblk, 1, BLK)
    out = pl.pallas_call(
        _affine_body,
        grid=(nblk,),
        in_specs=[
            pl.BlockSpec((1, 1, BLK), lambda i: (i, 0, 0)),
            pl.BlockSpec((E, D), lambda i: (0, 0)),
            pl.BlockSpec((E, D), lambda i: (0, 0)),
            pl.BlockSpec((BLK, D), lambda i: (i, 0)),
        ],
        out_specs=pl.BlockSpec((BLK, D), lambda i: (i, 0)),
        out_shape=jax.ShapeDtypeStruct((N, D), jnp.float32),
        compiler_params=pltpu.CompilerParams(
            dimension_semantics=("arbitrary",),
            vmem_limit_bytes=110 * 1024 * 1024,
        ),
    )(y3, s, b, x)
    ld = _logdet_kernel(s, y32)
    return out, ld
